# baseline (speedup 1.0000x reference)
import functools

import jax
import jax.numpy as jnp
from jax import lax
from jax.experimental import pallas as pl
from jax.experimental.pallas import tpu as pltpu
from jax.experimental.pallas import tpu_sc as plsc

N_INST = 10000
N_FG = 500000
CH = 400
G = CH // 16
NCHUNK = N_FG // CH
NW = 32
NPAIR = (NCHUNK // NW + 1 + 1) // 2


def _prep_body(pt_ref, tab_ref):
    h = pt_ref[6:7, :]
    tab_ref[...] = jnp.concatenate(
        [jnp.cos(h), jnp.sin(h), pt_ref[0:3, :], pt_ref[3:6, :] * 0.5],
        axis=0)


def _prep(pt):
    return pl.pallas_call(
        _prep_body,
        out_shape=jax.ShapeDtypeStruct((8, N_INST), jnp.float32),
    )(pt)


_COL_SRC = (
    0, 1, 2,
    4, 5, 8,
    4, 6, 8,
    4, 6, 7,
    4, 5, 7,
    3, 5, 8,
    3, 6, 8,
    3, 6, 7,
    3, 5, 7,
)


def _sc_body(table_hbm, fgx_hbm, fgy_hbm, fgz_hbm, idx_hbm, out_hbm,
             table_v, idx_vs, fgx_vs, fgy_vs, fgz_vs, out_vs,
             sem_ins, sem_outs):
    info = plsc.get_sparse_core_info()
    nw = info.num_cores * info.num_subcores
    wid = lax.axis_index("s") * info.num_cores + lax.axis_index("c")
    nmine = (NCHUNK - 1 - wid) // nw + 1

    pltpu.sync_copy(table_hbm, table_v)

    def in_dmas(i, b):
        base = (wid + i * nw) * CH
        sl = pl.ds(base, CH)
        return (
            pltpu.make_async_copy(idx_hbm.at[sl], idx_vs[b], sem_ins[b]),
            pltpu.make_async_copy(fgx_hbm.at[sl], fgx_vs[b], sem_ins[b]),
            pltpu.make_async_copy(fgy_hbm.at[sl], fgy_vs[b], sem_ins[b]),
            pltpu.make_async_copy(fgz_hbm.at[sl], fgz_vs[b], sem_ins[b]),
        )

    def out_dmas(i, b):
        base = (wid + i * nw) * CH
        return [
            pltpu.make_async_copy(
                out_vs[b].at[pl.ds(c * CH, CH)],
                out_hbm.at[pl.ds(c * N_FG + base, CH)],
                sem_outs[b])
            for c in range(27)
        ]

    def compute(b):
        idx_v, out_v = idx_vs[b], out_vs[b]
        fgx_v, fgy_v, fgz_v = fgx_vs[b], fgy_vs[b], fgz_vs[b]

        @plsc.parallel_loop(0, G, unroll=2)
        def _(g):
            off = g * 16
            pidx = idx_v[pl.ds(off, 16)]
            f = [plsc.load_gather(table_v, [pidx + c * N_INST])
                 for c in range(8)]
            cosv, sinv, cx, cy, cz, dx, dy, dz = f
            p1 = fgx_v[pl.ds(off, 16)]
            p2 = fgy_v[pl.ds(off, 16)]
            p3 = fgz_v[pl.ds(off, 16)]
            fibx = cosv * p1 - sinv * p2 + cx
            fiby = sinv * p1 + cosv * p2 + cy
            fibz = p3 + cz
            src = (fibx, fiby, fibz,
                   fibx + dx, fibx - dx,
                   fiby + dy, fiby - dy,
                   fibz + dz, fibz - dz)
            for c in range(27):
                out_v[pl.ds(c * CH + off, 16)] = src[_COL_SRC[c]]

    for b in (0, 1):
        for d in in_dmas(b, b):
            d.start()

    def pair_body(i2, _):
        for b in (0, 1):
            i = i2 * 2 + b

            @pl.when(i < nmine)
            def _():
                for d in in_dmas(i, b):
                    d.wait()

                @pl.when(i >= 2)
                def _():
                    for d in out_dmas(i - 2, b):
                        d.wait()

                compute(b)
                for d in out_dmas(i, b):
                    d.start()

                @pl.when(i + 2 < nmine)
                def _():
                    for d in in_dmas(i + 2, b):
                        d.start()
        return 0

    lax.fori_loop(0, NPAIR, pair_body, 0, unroll=False)

    for b in (0, 1):
        for d in out_dmas(b, b):
            d.wait()


@functools.partial(jax.jit, static_argnums=())
def _sc_main(table, fgx, fgy, fgz, idx):
    mesh = plsc.VectorSubcoreMesh(core_axis_name="c", subcore_axis_name="s")
    return pl.kernel(
        _sc_body,
        out_type=jax.ShapeDtypeStruct((27 * N_FG,), jnp.float32),
        mesh=mesh,
        compiler_params=pltpu.CompilerParams(
            needs_layout_passes=False, use_tc_tiling_on_sc=False),
        scratch_types=[
            pltpu.VMEM((8 * N_INST,), jnp.float32),
            [pltpu.VMEM((CH,), jnp.int32) for _ in range(2)],
            [pltpu.VMEM((CH,), jnp.float32) for _ in range(2)],
            [pltpu.VMEM((CH,), jnp.float32) for _ in range(2)],
            [pltpu.VMEM((CH,), jnp.float32) for _ in range(2)],
            [pltpu.VMEM((27 * CH,), jnp.float32) for _ in range(2)],
            [pltpu.SemaphoreType.DMA for _ in range(2)],
            [pltpu.SemaphoreType.DMA for _ in range(2)],
        ],
    )(table, fgx, fgy, fgz, idx)


def kernel(pred_boxes, fg, inst_bi_inv_indices):
    idx = inst_bi_inv_indices.astype(jnp.int32)
    table = _prep(pred_boxes.T).reshape(-1)
    out_t = _sc_main(table, fg[:, 1], fg[:, 2], fg[:, 3], idx)
    return out_t.reshape(27, N_FG).T


# speedup vs baseline: 2.6420x; 2.6420x over previous
import functools

import jax
import jax.numpy as jnp
from jax import lax
from jax.experimental import pallas as pl
from jax.experimental.pallas import tpu as pltpu
from jax.experimental.pallas import tpu_sc as plsc

N_INST = 10000
N_FG = 500000
CH = 400
G = CH // 16
NCHUNK = N_FG // CH
NW = 32
NPAIR = (NCHUNK // NW + 1 + 1) // 2


def _prep_body(pt_ref, tab_ref):
    h = pt_ref[6:7, :]
    tab_ref[...] = jnp.concatenate(
        [jnp.cos(h), jnp.sin(h), pt_ref[0:3, :], pt_ref[3:6, :] * 0.5],
        axis=0)


def _prep(pt):
    return pl.pallas_call(
        _prep_body,
        out_shape=jax.ShapeDtypeStruct((8, N_INST), jnp.float32),
    )(pt)


_COL_SRC = (
    0, 1, 2,
    4, 5, 8,
    4, 6, 8,
    4, 6, 7,
    4, 5, 7,
    3, 5, 8,
    3, 6, 8,
    3, 6, 7,
    3, 5, 7,
)


def _sc_body(table_hbm, fgx_hbm, fgy_hbm, fgz_hbm, idx_hbm, out_hbm,
             table_v, idx_vs, fgx_vs, fgy_vs, fgz_vs, out_vs,
             sem_ins, sem_outs):
    info = plsc.get_sparse_core_info()
    nw = info.num_cores * info.num_subcores
    wid = lax.axis_index("s") * info.num_cores + lax.axis_index("c")
    nmine = (NCHUNK - 1 - wid) // nw + 1

    pltpu.sync_copy(table_hbm, table_v)

    def in_dmas(i, b):
        base = (wid + i * nw) * CH
        sl = pl.ds(base, CH)
        return (
            pltpu.make_async_copy(idx_hbm.at[sl], idx_vs[b], sem_ins[b]),
            pltpu.make_async_copy(fgx_hbm.at[sl], fgx_vs[b], sem_ins[b]),
            pltpu.make_async_copy(fgy_hbm.at[sl], fgy_vs[b], sem_ins[b]),
            pltpu.make_async_copy(fgz_hbm.at[sl], fgz_vs[b], sem_ins[b]),
        )

    def out_dmas(i, b):
        base = (wid + i * nw) * CH
        return [
            pltpu.make_async_copy(
                out_vs[b], out_hbm.at[pl.ds(base * 27, CH * 27)],
                sem_outs[b])
        ]

    def compute(b):
        idx_v, out_v = idx_vs[b], out_vs[b]
        fgx_v, fgy_v, fgz_v = fgx_vs[b], fgy_vs[b], fgz_vs[b]

        lane = lax.iota(jnp.int32, 16)

        @plsc.parallel_loop(0, G, unroll=2)
        def _(g):
            off = g * 16
            pidx = idx_v[pl.ds(off, 16)]
            f = [plsc.load_gather(table_v, [pidx + c * N_INST])
                 for c in range(8)]
            cosv, sinv, cx, cy, cz, dx, dy, dz = f
            p1 = fgx_v[pl.ds(off, 16)]
            p2 = fgy_v[pl.ds(off, 16)]
            p3 = fgz_v[pl.ds(off, 16)]
            fibx = cosv * p1 - sinv * p2 + cx
            fiby = sinv * p1 + cosv * p2 + cy
            fibz = p3 + cz
            src = (fibx, fiby, fibz,
                   fibx + dx, fibx - dx,
                   fiby + dy, fiby - dy,
                   fibz + dz, fibz - dz)
            rows27 = (off + lane) * 27
            for c in range(27):
                plsc.store_scatter(out_v, [rows27 + c], src[_COL_SRC[c]])

    for b in (0, 1):
        for d in in_dmas(b, b):
            d.start()

    def pair_body(i2, _):
        for b in (0, 1):
            i = i2 * 2 + b

            @pl.when(i < nmine)
            def _():
                for d in in_dmas(i, b):
                    d.wait()

                @pl.when(i >= 2)
                def _():
                    for d in out_dmas(i - 2, b):
                        d.wait()

                compute(b)
                for d in out_dmas(i, b):
                    d.start()

                @pl.when(i + 2 < nmine)
                def _():
                    for d in in_dmas(i + 2, b):
                        d.start()
        return 0

    lax.fori_loop(0, NPAIR, pair_body, 0, unroll=False)

    for b in (0, 1):
        for d in out_dmas(b, b):
            d.wait()


@functools.partial(jax.jit, static_argnums=())
def _sc_main(table, fgx, fgy, fgz, idx):
    mesh = plsc.VectorSubcoreMesh(core_axis_name="c", subcore_axis_name="s")
    return pl.kernel(
        _sc_body,
        out_type=jax.ShapeDtypeStruct((27 * N_FG,), jnp.float32),
        mesh=mesh,
        compiler_params=pltpu.CompilerParams(
            needs_layout_passes=False, use_tc_tiling_on_sc=False),
        scratch_types=[
            pltpu.VMEM((8 * N_INST,), jnp.float32),
            [pltpu.VMEM((CH,), jnp.int32) for _ in range(2)],
            [pltpu.VMEM((CH,), jnp.float32) for _ in range(2)],
            [pltpu.VMEM((CH,), jnp.float32) for _ in range(2)],
            [pltpu.VMEM((CH,), jnp.float32) for _ in range(2)],
            [pltpu.VMEM((27 * CH,), jnp.float32) for _ in range(2)],
            [pltpu.SemaphoreType.DMA for _ in range(2)],
            [pltpu.SemaphoreType.DMA for _ in range(2)],
        ],
    )(table, fgx, fgy, fgz, idx)


def kernel(pred_boxes, fg, inst_bi_inv_indices):
    idx = inst_bi_inv_indices.astype(jnp.int32)
    table = _prep(pred_boxes.T).reshape(-1)
    out = _sc_main(table, fg[:, 1], fg[:, 2], fg[:, 3], idx)
    return out.reshape(N_FG, 27)
